# baseline (speedup 1.0000x reference)
import jax
import jax.numpy as jnp
from jax import lax
from jax.experimental import pallas as pl
from jax.experimental.pallas import tpu as pltpu
from jax.experimental.pallas import tpu_sc as plsc

_N = 10000
_D = 16
_S = 8
_DIN = 256
_DH = _DIN // 2
_DOUT = 512
_NC = 2
_NS = 16
_NW = _NC * _NS
_B = 8
_CH = 40
_PERW = _B * _CH
_NPAD = _NW * _PERW


def _agg_body(nbr_hbm, x_hbm, s_out, c_out, idx_v, rows_v, sv, cv,
              sem0, sem1, sem2, sem3, osem0, osem1):
    wid = lax.axis_index("s") * _NC + lax.axis_index("c")
    base = wid * _PERW
    pltpu.sync_copy(
        nbr_hbm.at[pl.ds(pl.multiple_of(base * _D, 128), _PERW * _D)], idx_v)

    gsems = (sem0, sem1, sem2, sem3)

    def gather(g, b):
        return pltpu.make_async_copy(
            x_hbm.at[idx_v.at[pl.ds(g * (_B * _D), _B * _D)]],
            rows_v.at[b], gsems[b])

    def out_copies(g, b):
        osem = osem0 if b == 0 else osem1
        node0 = base + g * _B
        return (
            pltpu.make_async_copy(sv.at[b], s_out.at[pl.ds(node0, _B)], osem),
            pltpu.make_async_copy(cv.at[b], c_out.at[pl.ds(node0, _B)], osem),
        )

    for p in range(3):
        gather(p, p).start()
    scale = jnp.bfloat16(1.0 / _S)

    def outer(i, carry):
        for q in range(4):
            g = i * 4 + q
            b = q
            ob = q % 2
            nxt = g + 3

            @pl.when(nxt < _CH)
            def _():
                gather(nxt, (q + 3) % 4).start()

            gather(g, b).wait()

            @pl.when(g >= 2)
            def _():
                for cp in out_copies(g - 2, ob):
                    cp.wait()

            rv = rows_v.at[b].bitcast(jnp.bfloat16)
            svv = sv.at[ob].bitcast(jnp.bfloat16)
            cvv = cv.at[ob].bitcast(jnp.bfloat16)

            @plsc.parallel_loop(0, _B, step=2)
            def _(n):
                rn = rv.at[pl.ds(pl.multiple_of(n * 2 * _D, 4 * _D), 4 * _D)]
                svp = svv.at[pl.ds(pl.multiple_of(n * 2, 4), 4)]
                cvp = cvv.at[pl.ds(pl.multiple_of(n * 2, 4), 4)]
                for u in range(2):
                    ro = u * 2 * _D
                    us = pl.ds(2 * u, 2)
                    for c in range(_DIN // 32):
                        slc = pl.ds(c * 16, 16)

                        def ld(k):
                            return rn[pl.ds(ro + 2 * k, 2), slc]

                        s01 = ld(0) + ld(1)
                        s23 = ld(2) + ld(3)
                        s45 = ld(4) + ld(5)
                        s67 = ld(6) + ld(7)
                        svp[us, slc] = ((s01 + s23) + (s45 + s67)) * scale
                        c01 = ld(8) + ld(9)
                        c23 = ld(10) + ld(11)
                        c45 = ld(12) + ld(13)
                        c67 = ld(14) + ld(15)
                        cvp[us, slc] = ((c01 + c23) + (c45 + c67)) * scale

            for cp in out_copies(g, ob):
                cp.start()
        return carry

    lax.fori_loop(0, _CH // 4, outer, 0)
    for g in (_CH - 2, _CH - 1):
        for cp in out_copies(g, g % 2):
            cp.wait()


_agg = pl.kernel(
    _agg_body,
    out_type=[
        jax.ShapeDtypeStruct((_NPAD, _DH), jnp.int32),
        jax.ShapeDtypeStruct((_NPAD, _DH), jnp.int32),
    ],
    mesh=plsc.VectorSubcoreMesh(
        core_axis_name="c", subcore_axis_name="s", num_cores=_NC,
        num_subcores=_NS),
    scratch_types=[
        pltpu.VMEM((_PERW * _D,), jnp.int32),
        pltpu.VMEM((4, _B * _D, _DH), jnp.int32),
        pltpu.VMEM((2, _B, _DH), jnp.int32),
        pltpu.VMEM((2, _B, _DH), jnp.int32),
        pltpu.SemaphoreType.DMA,
        pltpu.SemaphoreType.DMA,
        pltpu.SemaphoreType.DMA,
        pltpu.SemaphoreType.DMA,
        pltpu.SemaphoreType.DMA,
        pltpu.SemaphoreType.DMA,
    ],
)


_BM = 1000


def _pack_body(x_ref, o_ref):
    bits = lax.bitcast_convert_type(x_ref[...], jnp.int32)
    rnd = ((bits >> 16) & 1) + 0x7FFF
    bf = ((bits + rnd) >> 16) & 0xFFFF
    o_ref[...] = bf[:, :_DH] | (bf[:, _DH:] << 16)


def _pack(x):
    return pl.pallas_call(
        _pack_body,
        grid=(_N // _BM,),
        in_specs=[pl.BlockSpec((_BM, _DIN), lambda i: (i, 0))],
        out_specs=pl.BlockSpec((_BM, _DH), lambda i: (i, 0)),
        out_shape=jax.ShapeDtypeStruct((_N, _DH), jnp.int32),
    )(x)


def _unpack_lo_hi(v):
    lo = lax.bitcast_convert_type(v << 16, jnp.float32)
    hi = lax.bitcast_convert_type(v & jnp.int32(-65536), jnp.float32)
    return lo.astype(jnp.bfloat16), hi.astype(jnp.bfloat16)


def _gemm_body(x_ref, s_ref, c_ref, ws_lo, ws_hi, wn_lo, wn_hi, wc_lo,
               wc_hi, bs, bn, bc, o_ref):
    x_lo, x_hi = _unpack_lo_hi(x_ref[...])
    acc = jnp.dot(x_lo, ws_lo[...], preferred_element_type=jnp.float32)
    acc += jnp.dot(x_hi, ws_hi[...], preferred_element_type=jnp.float32)
    s_lo, s_hi = _unpack_lo_hi(s_ref[...])
    acc += jnp.dot(s_lo, wn_lo[...], preferred_element_type=jnp.float32)
    acc += jnp.dot(s_hi, wn_hi[...], preferred_element_type=jnp.float32)
    c_lo, c_hi = _unpack_lo_hi(c_ref[...])
    acc += jnp.dot(c_lo, wc_lo[...], preferred_element_type=jnp.float32)
    acc += jnp.dot(c_hi, wc_hi[...], preferred_element_type=jnp.float32)
    o_ref[...] = acc + (bs[...] + bn[...] + bc[...])


def _fused_gemm(xi, s_agg, c_agg, ws, wn, wc, bs, bn, bc):
    a_spec = pl.BlockSpec((_BM, _DH), lambda i: (i, 0))
    wh_spec = pl.BlockSpec((_DH, _DOUT), lambda i: (0, 0))
    b_spec = pl.BlockSpec((1, _DOUT), lambda i: (0, 0))
    return pl.pallas_call(
        _gemm_body,
        grid=(_N // _BM,),
        in_specs=[a_spec, a_spec, a_spec] + [wh_spec] * 6
                 + [b_spec, b_spec, b_spec],
        out_specs=pl.BlockSpec((_BM, _DOUT), lambda i: (i, 0)),
        out_shape=jax.ShapeDtypeStruct((_N, _DOUT), jnp.float32),
    )(xi, s_agg, c_agg, ws[:_DH], ws[_DH:], wn[:_DH], wn[_DH:],
      wc[:_DH], wc[_DH:],
      bs.reshape(1, -1), bn.reshape(1, -1), bc.reshape(1, -1))


def kernel(x, neighbors, W_self, b_self, W_neigh, b_neigh, W_coarsen,
           b_coarsen):
    nbr_pad = jnp.pad(neighbors, ((0, _NPAD - _N), (0, 0)))
    xi = _pack(x)
    s_agg, c_agg = _agg(nbr_pad.reshape(-1), xi)
    return _fused_gemm(
        xi, s_agg, c_agg, W_self.astype(jnp.bfloat16),
        W_neigh.astype(jnp.bfloat16), W_coarsen.astype(jnp.bfloat16),
        b_self, b_neigh, b_coarsen)


# speedup vs baseline: 1.0123x; 1.0123x over previous
import jax
import jax.numpy as jnp
from jax import lax
from jax.experimental import pallas as pl
from jax.experimental.pallas import tpu as pltpu
from jax.experimental.pallas import tpu_sc as plsc

_N = 10000
_D = 16
_S = 8
_DIN = 256
_DH = _DIN // 2
_DOUT = 512
_NC = 2
_NS = 16
_NW = _NC * _NS
_B = 8
_CH = 40
_PERW = _B * _CH
_NPAD = _NW * _PERW


def _agg_body(nbr_hbm, x_hbm, s_out, c_out, idx_v, rows_v, sv, cv,
              sem0, sem1, sem2, sem3, osem0, osem1):
    wid = lax.axis_index("s") * _NC + lax.axis_index("c")
    base = wid * _PERW
    pltpu.sync_copy(
        nbr_hbm.at[pl.ds(pl.multiple_of(base * _D, 128), _PERW * _D)], idx_v)

    gsems = (sem0, sem1, sem2, sem3)

    def gather(g, b):
        return pltpu.make_async_copy(
            x_hbm.at[idx_v.at[pl.ds(g * (_B * _D), _B * _D)]],
            rows_v.at[b], gsems[b])

    def out_copies(g, b):
        osem = osem0 if b == 0 else osem1
        node0 = base + g * _B
        return (
            pltpu.make_async_copy(sv.at[b], s_out.at[pl.ds(node0, _B)], osem),
            pltpu.make_async_copy(cv.at[b], c_out.at[pl.ds(node0, _B)], osem),
        )

    for p in range(3):
        gather(p, p).start()
    scale = jnp.bfloat16(1.0 / _S)

    def outer(i, carry):
        for q in range(4):
            g = i * 4 + q
            b = q
            ob = q % 2
            nxt = g + 3

            @pl.when(nxt < _CH)
            def _():
                gather(nxt, (q + 3) % 4).start()

            gather(g, b).wait()

            @pl.when(g >= 2)
            def _():
                for cp in out_copies(g - 2, ob):
                    cp.wait()

            rv = rows_v.at[b].bitcast(jnp.bfloat16)
            svv = sv.at[ob].bitcast(jnp.bfloat16)
            cvv = cv.at[ob].bitcast(jnp.bfloat16)

            @plsc.parallel_loop(0, _B, step=2, unroll=2)
            def _(n):
                rn = rv.at[pl.ds(pl.multiple_of(n * 2 * _D, 4 * _D), 4 * _D)]
                svp = svv.at[pl.ds(pl.multiple_of(n * 2, 4), 4)]
                cvp = cvv.at[pl.ds(pl.multiple_of(n * 2, 4), 4)]
                for u in range(2):
                    ro = u * 2 * _D
                    us = pl.ds(2 * u, 2)
                    for c in range(_DIN // 32):
                        slc = pl.ds(c * 16, 16)

                        def ld(k):
                            return rn[pl.ds(ro + 2 * k, 2), slc]

                        s01 = ld(0) + ld(1)
                        s23 = ld(2) + ld(3)
                        s45 = ld(4) + ld(5)
                        s67 = ld(6) + ld(7)
                        svp[us, slc] = ((s01 + s23) + (s45 + s67)) * scale
                        c01 = ld(8) + ld(9)
                        c23 = ld(10) + ld(11)
                        c45 = ld(12) + ld(13)
                        c67 = ld(14) + ld(15)
                        cvp[us, slc] = ((c01 + c23) + (c45 + c67)) * scale

            for cp in out_copies(g, ob):
                cp.start()
        return carry

    lax.fori_loop(0, _CH // 4, outer, 0)
    for g in (_CH - 2, _CH - 1):
        for cp in out_copies(g, g % 2):
            cp.wait()


_agg = pl.kernel(
    _agg_body,
    out_type=[
        jax.ShapeDtypeStruct((_NPAD, _DH), jnp.int32),
        jax.ShapeDtypeStruct((_NPAD, _DH), jnp.int32),
    ],
    mesh=plsc.VectorSubcoreMesh(
        core_axis_name="c", subcore_axis_name="s", num_cores=_NC,
        num_subcores=_NS),
    scratch_types=[
        pltpu.VMEM((_PERW * _D,), jnp.int32),
        pltpu.VMEM((4, _B * _D, _DH), jnp.int32),
        pltpu.VMEM((2, _B, _DH), jnp.int32),
        pltpu.VMEM((2, _B, _DH), jnp.int32),
        pltpu.SemaphoreType.DMA,
        pltpu.SemaphoreType.DMA,
        pltpu.SemaphoreType.DMA,
        pltpu.SemaphoreType.DMA,
        pltpu.SemaphoreType.DMA,
        pltpu.SemaphoreType.DMA,
    ],
)


_BM = 1000


def _pack_body(x_ref, o_ref):
    bits = lax.bitcast_convert_type(x_ref[...], jnp.int32)
    rnd = ((bits >> 16) & 1) + 0x7FFF
    bf = ((bits + rnd) >> 16) & 0xFFFF
    o_ref[...] = bf[:, :_DH] | (bf[:, _DH:] << 16)


def _pack(x):
    return pl.pallas_call(
        _pack_body,
        grid=(_N // _BM,),
        in_specs=[pl.BlockSpec((_BM, _DIN), lambda i: (i, 0))],
        out_specs=pl.BlockSpec((_BM, _DH), lambda i: (i, 0)),
        out_shape=jax.ShapeDtypeStruct((_N, _DH), jnp.int32),
    )(x)


def _unpack_lo_hi(v):
    lo = lax.bitcast_convert_type(v << 16, jnp.float32)
    hi = lax.bitcast_convert_type(v & jnp.int32(-65536), jnp.float32)
    return lo.astype(jnp.bfloat16), hi.astype(jnp.bfloat16)


def _gemm_body(x_ref, s_ref, c_ref, ws_lo, ws_hi, wn_lo, wn_hi, wc_lo,
               wc_hi, bs, bn, bc, o_ref):
    x_lo, x_hi = _unpack_lo_hi(x_ref[...])
    acc = jnp.dot(x_lo, ws_lo[...], preferred_element_type=jnp.float32)
    acc += jnp.dot(x_hi, ws_hi[...], preferred_element_type=jnp.float32)
    s_lo, s_hi = _unpack_lo_hi(s_ref[...])
    acc += jnp.dot(s_lo, wn_lo[...], preferred_element_type=jnp.float32)
    acc += jnp.dot(s_hi, wn_hi[...], preferred_element_type=jnp.float32)
    c_lo, c_hi = _unpack_lo_hi(c_ref[...])
    acc += jnp.dot(c_lo, wc_lo[...], preferred_element_type=jnp.float32)
    acc += jnp.dot(c_hi, wc_hi[...], preferred_element_type=jnp.float32)
    o_ref[...] = acc + (bs[...] + bn[...] + bc[...])


def _fused_gemm(xi, s_agg, c_agg, ws, wn, wc, bs, bn, bc):
    a_spec = pl.BlockSpec((_BM, _DH), lambda i: (i, 0))
    wh_spec = pl.BlockSpec((_DH, _DOUT), lambda i: (0, 0))
    b_spec = pl.BlockSpec((1, _DOUT), lambda i: (0, 0))
    return pl.pallas_call(
        _gemm_body,
        grid=(_N // _BM,),
        in_specs=[a_spec, a_spec, a_spec] + [wh_spec] * 6
                 + [b_spec, b_spec, b_spec],
        out_specs=pl.BlockSpec((_BM, _DOUT), lambda i: (i, 0)),
        out_shape=jax.ShapeDtypeStruct((_N, _DOUT), jnp.float32),
    )(xi, s_agg, c_agg, ws[:_DH], ws[_DH:], wn[:_DH], wn[_DH:],
      wc[:_DH], wc[_DH:],
      bs.reshape(1, -1), bn.reshape(1, -1), bc.reshape(1, -1))


def kernel(x, neighbors, W_self, b_self, W_neigh, b_neigh, W_coarsen,
           b_coarsen):
    nbr_pad = jnp.pad(neighbors, ((0, _NPAD - _N), (0, 0)))
    xi = _pack(x)
    s_agg, c_agg = _agg(nbr_pad.reshape(-1), xi)
    return _fused_gemm(
        xi, s_agg, c_agg, W_self.astype(jnp.bfloat16),
        W_neigh.astype(jnp.bfloat16), W_coarsen.astype(jnp.bfloat16),
        b_self, b_neigh, b_coarsen)
